# baseline (speedup 1.0000x reference)
import functools

import jax
import jax.numpy as jnp
from jax import lax
from jax.experimental import pallas as pl
from jax.experimental.pallas import tpu as pltpu
from jax.experimental.pallas import tpu_sc as plsc

NW = 32


def _make_gather(n_rows, d_model, chunk, dtype):
    mesh = plsc.VectorSubcoreMesh(core_axis_name="c", subcore_axis_name="s")
    rows_per_w = n_rows // NW
    nc = rows_per_w // chunk

    @functools.partial(
        pl.kernel,
        out_type=jax.ShapeDtypeStruct((n_rows, d_model), dtype),
        mesh=mesh,
        scratch_types=[
            pltpu.VMEM((rows_per_w,), jnp.int32),
            pltpu.VMEM((2, chunk, d_model), dtype),
            pltpu.SemaphoreType.DMA,
            pltpu.SemaphoreType.DMA,
            pltpu.SemaphoreType.DMA,
            pltpu.SemaphoreType.DMA,
        ],
    )
    def gather_k(x_hbm, rows_hbm, out_hbm, idx_v, buf_v, g0, g1, s0, s1):
        wid = lax.axis_index("s") * 2 + lax.axis_index("c")
        base = wid * rows_per_w
        gsem = (g0, g1)
        ssem = (s0, s1)
        pltpu.sync_copy(rows_hbm.at[pl.ds(base, rows_per_w)], idx_v)

        gh = [None] * nc
        sh = [None] * nc
        gh[0] = pltpu.async_copy(
            x_hbm.at[idx_v.at[pl.ds(0, chunk)]], buf_v.at[0], gsem[0]
        )
        for c in range(nc):
            if c + 1 < nc:
                b = (c + 1) % 2
                if c - 1 >= 0:
                    sh[c - 1].wait()
                gh[c + 1] = pltpu.async_copy(
                    x_hbm.at[idx_v.at[pl.ds((c + 1) * chunk, chunk)]],
                    buf_v.at[b],
                    gsem[b],
                )
            gh[c].wait()
            sh[c] = pltpu.async_copy(
                buf_v.at[c % 2], out_hbm.at[pl.ds(base + c * chunk, chunk)],
                ssem[c % 2],
            )
        sh[nc - 2].wait()
        sh[nc - 1].wait()

    return gather_k


def _make_combine(n_tokens, d_model, chunk):
    mesh = plsc.VectorSubcoreMesh(core_axis_name="c", subcore_axis_name="s")
    tok_per_w = n_tokens // NW
    lanes_per_row = d_model // 16

    @functools.partial(
        pl.kernel,
        out_type=jax.ShapeDtypeStruct((n_tokens, d_model), jnp.float32),
        mesh=mesh,
        scratch_types=[
            pltpu.VMEM((chunk,), jnp.int32),
            pltpu.VMEM((chunk,), jnp.int32),
            pltpu.VMEM((chunk, d_model), jnp.float32),
            pltpu.VMEM((chunk, d_model), jnp.float32),
            pltpu.SemaphoreType.DMA,
        ],
    )
    def combine_k(y_hbm, r0_hbm, r1_hbm, out_hbm, i0_v, i1_v, a_v, b_v, sem):
        wid = lax.axis_index("s") * 2 + lax.axis_index("c")
        base = wid * tok_per_w

        def body(i, carry):
            off = base + i * chunk
            pltpu.sync_copy(r0_hbm.at[pl.ds(off, chunk)], i0_v)
            pltpu.sync_copy(r1_hbm.at[pl.ds(off, chunk)], i1_v)
            pltpu.async_copy(y_hbm.at[i0_v], a_v, sem).wait()
            pltpu.async_copy(y_hbm.at[i1_v], b_v, sem).wait()

            def row_add(r, c2):
                def col_add(c, c3):
                    for u in range(8):
                        sl = pl.ds((c * 8 + u) * 16, 16)
                        a_v[r, sl] = a_v[r, sl] + b_v[r, sl]
                    return c3

                lax.fori_loop(0, lanes_per_row // 8, col_add, 0)
                return c2

            lax.fori_loop(0, chunk, row_add, 0)
            pltpu.sync_copy(a_v, out_hbm.at[pl.ds(off, chunk)])
            return carry

        lax.fori_loop(0, tok_per_w // chunk, body, 0)

    return combine_k


def _gmm_body(te_ref, xg_ref, wg_ref, wu_ref, wd_ref, wrow_ref, y_ref):
    j = pl.program_id(1)
    x = xg_ref[...]
    g = jnp.dot(x, wg_ref[0].T, preferred_element_type=jnp.float32)
    u = jnp.dot(x, wu_ref[0].T, preferred_element_type=jnp.float32)
    h = ((g * jax.nn.sigmoid(g)) * u).astype(jnp.bfloat16)
    yj = jnp.dot(h, wd_ref[0].T, preferred_element_type=jnp.float32)
    yj = yj * wrow_ref[0, 0, :][:, None]

    @pl.when(j == 0)
    def _():
        y_ref[...] = jnp.zeros_like(y_ref)

    y_ref[...] += yj


def _make_gmm(n_rows, d_model, d_expert, n_experts, blk_m, blk_n):
    nt = n_rows // blk_m
    nb = d_expert // blk_n
    grid_spec = pltpu.PrefetchScalarGridSpec(
        num_scalar_prefetch=1,
        grid=(nt, nb),
        in_specs=[
            pl.BlockSpec((blk_m, d_model), lambda i, j, te: (i, 0)),
            pl.BlockSpec((1, blk_n, d_model), lambda i, j, te: (te[i], j, 0)),
            pl.BlockSpec((1, blk_n, d_model), lambda i, j, te: (te[i], j, 0)),
            pl.BlockSpec((1, d_model, blk_n), lambda i, j, te: (te[i], 0, j)),
            pl.BlockSpec((1, 1, blk_m), lambda i, j, te: (i, 0, 0)),
        ],
        out_specs=pl.BlockSpec((blk_m, d_model), lambda i, j, te: (i, 0)),
    )
    return pl.pallas_call(
        _gmm_body,
        grid_spec=grid_spec,
        out_shape=jax.ShapeDtypeStruct((n_rows, d_model), jnp.float32),
        compiler_params=pltpu.CompilerParams(
            dimension_semantics=("arbitrary", "arbitrary"),
        ),
    )


def kernel(x, routing_weights, expert_indices, w_gate, w_up, w_down):
    batch, seq_len, d_model = x.shape
    top_k = expert_indices.shape[-1]
    n_experts, d_expert, _ = w_gate.shape
    n_tokens = batch * seq_len
    n_assign = n_tokens * top_k

    blk_m = 512
    blk_n = 512
    n_rows = n_assign + n_experts * blk_m

    x_flat = x.reshape(n_tokens, d_model)
    e_flat = expert_indices.reshape(n_assign).astype(jnp.int32)
    w_flat = routing_weights.reshape(n_assign).astype(jnp.float32)

    order = jnp.argsort(e_flat)
    e_sorted = jnp.take(e_flat, order)
    counts = jnp.bincount(e_flat, length=n_experts)
    starts = jnp.cumsum(counts) - counts
    pc = ((counts + blk_m - 1) // blk_m) * blk_m
    padded_starts = jnp.cumsum(pc) - pc
    p = jnp.arange(n_assign, dtype=jnp.int32)
    row_sorted = (padded_starts[e_sorted] + (p - starts[e_sorted])).astype(jnp.int32)
    token_for_sorted = (order // top_k).astype(jnp.int32)
    token_row = jnp.zeros((n_rows,), jnp.int32).at[row_sorted].set(token_for_sorted)
    w_row = jnp.zeros((n_rows,), jnp.float32).at[row_sorted].set(jnp.take(w_flat, order))
    nt = n_rows // blk_m
    tile_start = jnp.arange(nt, dtype=jnp.int32) * blk_m
    pcum = jnp.cumsum(pc)
    tile_expert = jnp.minimum(
        jnp.searchsorted(pcum, tile_start, side="right"), n_experts - 1
    ).astype(jnp.int32)
    row_by_a = jnp.zeros((n_assign,), jnp.int32).at[order].set(row_sorted)
    r0 = row_by_a[0::top_k]
    r1 = row_by_a[1::top_k]

    g_chunk = 64
    x_bf = x_flat.astype(jnp.bfloat16)
    x_i = lax.bitcast_convert_type(
        x_bf.reshape(n_tokens, d_model // 2, 2), jnp.int32
    )
    xg_i = _make_gather(n_rows, d_model // 2, g_chunk, jnp.int32)(x_i, token_row)
    xg = lax.bitcast_convert_type(xg_i, jnp.bfloat16).reshape(n_rows, d_model)

    w_row3 = w_row.reshape(nt, 1, blk_m)
    y = _make_gmm(n_rows, d_model, d_expert, n_experts, blk_m, blk_n)(
        tile_expert,
        xg,
        w_gate.astype(jnp.bfloat16),
        w_up.astype(jnp.bfloat16),
        w_down.astype(jnp.bfloat16),
        w_row3,
    )

    out = _make_combine(n_tokens, d_model, chunk=32)(y, r0, r1)
    return out.reshape(batch, seq_len, d_model)


# speedup vs baseline: 1.5127x; 1.5127x over previous
import functools

import jax
import jax.numpy as jnp
from jax import lax
from jax.experimental import pallas as pl
from jax.experimental.pallas import tpu as pltpu
from jax.experimental.pallas import tpu_sc as plsc

NW = 32


def _make_gather(n_rows, d_model, chunk, dtype):
    mesh = plsc.VectorSubcoreMesh(core_axis_name="c", subcore_axis_name="s")
    rows_per_w = n_rows // NW
    nc = rows_per_w // chunk

    @functools.partial(
        pl.kernel,
        out_type=jax.ShapeDtypeStruct((n_rows, d_model), dtype),
        mesh=mesh,
        scratch_types=[
            pltpu.VMEM((rows_per_w,), jnp.int32),
            pltpu.VMEM((2, chunk, d_model), dtype),
            pltpu.SemaphoreType.DMA,
            pltpu.SemaphoreType.DMA,
            pltpu.SemaphoreType.DMA,
            pltpu.SemaphoreType.DMA,
        ],
    )
    def gather_k(x_hbm, rows_hbm, out_hbm, idx_v, buf_v, g0, g1, s0, s1):
        wid = lax.axis_index("s") * 2 + lax.axis_index("c")
        base = wid * rows_per_w
        gsem = (g0, g1)
        ssem = (s0, s1)
        pltpu.sync_copy(rows_hbm.at[pl.ds(base, rows_per_w)], idx_v)

        gh = [None] * nc
        sh = [None] * nc
        gh[0] = pltpu.async_copy(
            x_hbm.at[idx_v.at[pl.ds(0, chunk)]], buf_v.at[0], gsem[0]
        )
        for c in range(nc):
            if c + 1 < nc:
                b = (c + 1) % 2
                if c - 1 >= 0:
                    sh[c - 1].wait()
                gh[c + 1] = pltpu.async_copy(
                    x_hbm.at[idx_v.at[pl.ds((c + 1) * chunk, chunk)]],
                    buf_v.at[b],
                    gsem[b],
                )
            gh[c].wait()
            sh[c] = pltpu.async_copy(
                buf_v.at[c % 2], out_hbm.at[pl.ds(base + c * chunk, chunk)],
                ssem[c % 2],
            )
        sh[nc - 2].wait()
        sh[nc - 1].wait()

    return gather_k


def _make_combine(n_tokens, d_model, chunk):
    mesh = plsc.VectorSubcoreMesh(core_axis_name="c", subcore_axis_name="s")
    tok_per_w = n_tokens // NW
    lanes_per_row = d_model // 16

    @functools.partial(
        pl.kernel,
        out_type=jax.ShapeDtypeStruct((n_tokens, d_model), jnp.float32),
        mesh=mesh,
        scratch_types=[
            pltpu.VMEM((chunk,), jnp.int32),
            pltpu.VMEM((chunk,), jnp.int32),
            pltpu.VMEM((chunk, d_model), jnp.float32),
            pltpu.VMEM((chunk, d_model), jnp.float32),
            pltpu.SemaphoreType.DMA,
        ],
    )
    def combine_k(y_hbm, r0_hbm, r1_hbm, out_hbm, i0_v, i1_v, a_v, b_v, sem):
        wid = lax.axis_index("s") * 2 + lax.axis_index("c")
        base = wid * tok_per_w

        def body(i, carry):
            off = base + i * chunk
            pltpu.sync_copy(r0_hbm.at[pl.ds(off, chunk)], i0_v)
            pltpu.sync_copy(r1_hbm.at[pl.ds(off, chunk)], i1_v)
            pltpu.async_copy(y_hbm.at[i0_v], a_v, sem).wait()
            pltpu.async_copy(y_hbm.at[i1_v], b_v, sem).wait()

            def row_add(r, c2):
                def col_add(c, c3):
                    for u in range(8):
                        sl = pl.ds((c * 8 + u) * 16, 16)
                        a_v[r, sl] = a_v[r, sl] + b_v[r, sl]
                    return c3

                lax.fori_loop(0, lanes_per_row // 8, col_add, 0)
                return c2

            lax.fori_loop(0, chunk, row_add, 0)
            pltpu.sync_copy(a_v, out_hbm.at[pl.ds(off, chunk)])
            return carry

        lax.fori_loop(0, tok_per_w // chunk, body, 0)

    return combine_k


def _gmm_body(te_ref, xg_ref, wg_ref, wu_ref, wd_ref, wrow_ref, y_ref):
    j = pl.program_id(1)
    x = xg_ref[...]
    g = jnp.dot(x, wg_ref[0].T, preferred_element_type=jnp.float32)
    u = jnp.dot(x, wu_ref[0].T, preferred_element_type=jnp.float32)
    h = (g * jax.nn.sigmoid(g)) * u
    yj = jnp.dot(h, wd_ref[0].T, preferred_element_type=jnp.float32)
    yj = yj * wrow_ref[0, 0, :][:, None]

    @pl.when(j == 0)
    def _():
        y_ref[...] = jnp.zeros_like(y_ref)

    y_ref[...] += yj


def _make_gmm(n_rows, d_model, d_expert, n_experts, blk_m, blk_n):
    nt = n_rows // blk_m
    nb = d_expert // blk_n
    grid_spec = pltpu.PrefetchScalarGridSpec(
        num_scalar_prefetch=1,
        grid=(nt, nb),
        in_specs=[
            pl.BlockSpec((blk_m, d_model), lambda i, j, te: (i, 0)),
            pl.BlockSpec((1, blk_n, d_model), lambda i, j, te: (te[i], j, 0)),
            pl.BlockSpec((1, blk_n, d_model), lambda i, j, te: (te[i], j, 0)),
            pl.BlockSpec((1, d_model, blk_n), lambda i, j, te: (te[i], 0, j)),
            pl.BlockSpec((1, 1, blk_m), lambda i, j, te: (i, 0, 0)),
        ],
        out_specs=pl.BlockSpec((blk_m, d_model), lambda i, j, te: (i, 0)),
    )
    return pl.pallas_call(
        _gmm_body,
        grid_spec=grid_spec,
        out_shape=jax.ShapeDtypeStruct((n_rows, d_model), jnp.float32),
        compiler_params=pltpu.CompilerParams(
            dimension_semantics=("arbitrary", "arbitrary"),
        ),
    )


def kernel(x, routing_weights, expert_indices, w_gate, w_up, w_down):
    batch, seq_len, d_model = x.shape
    top_k = expert_indices.shape[-1]
    n_experts, d_expert, _ = w_gate.shape
    n_tokens = batch * seq_len
    n_assign = n_tokens * top_k

    blk_m = 512
    blk_n = 512
    n_rows = n_assign + n_experts * blk_m

    x_flat = x.reshape(n_tokens, d_model)
    e_flat = expert_indices.reshape(n_assign).astype(jnp.int32)
    w_flat = routing_weights.reshape(n_assign).astype(jnp.float32)

    order = jnp.argsort(e_flat)
    e_sorted = jnp.take(e_flat, order)
    counts = jnp.bincount(e_flat, length=n_experts)
    starts = jnp.cumsum(counts) - counts
    pc = ((counts + blk_m - 1) // blk_m) * blk_m
    padded_starts = jnp.cumsum(pc) - pc
    p = jnp.arange(n_assign, dtype=jnp.int32)
    row_sorted = (padded_starts[e_sorted] + (p - starts[e_sorted])).astype(jnp.int32)
    token_for_sorted = (order // top_k).astype(jnp.int32)
    token_row = jnp.zeros((n_rows,), jnp.int32).at[row_sorted].set(token_for_sorted)
    w_row = jnp.zeros((n_rows,), jnp.float32).at[row_sorted].set(jnp.take(w_flat, order))
    nt = n_rows // blk_m
    tile_start = jnp.arange(nt, dtype=jnp.int32) * blk_m
    pcum = jnp.cumsum(pc)
    tile_expert = jnp.minimum(
        jnp.searchsorted(pcum, tile_start, side="right"), n_experts - 1
    ).astype(jnp.int32)
    row_by_a = jnp.zeros((n_assign,), jnp.int32).at[order].set(row_sorted)
    r0 = row_by_a[0::top_k]
    r1 = row_by_a[1::top_k]

    xg = _make_gather(n_rows, d_model, 48, jnp.float32)(x_flat, token_row)

    w_row3 = w_row.reshape(nt, 1, blk_m)
    y = _make_gmm(n_rows, d_model, d_expert, n_experts, blk_m, blk_n)(
        tile_expert, xg, w_gate, w_up, w_down, w_row3
    )

    out = _make_combine(n_tokens, d_model, chunk=32)(y, r0, r1)
    return out.reshape(batch, seq_len, d_model)


# speedup vs baseline: 1.5826x; 1.0462x over previous
import functools

import jax
import jax.numpy as jnp
from jax import lax
from jax.experimental import pallas as pl
from jax.experimental.pallas import tpu as pltpu
from jax.experimental.pallas import tpu_sc as plsc

NW = 32


def _make_gather(n_rows, d_model, chunk, dtype):
    mesh = plsc.VectorSubcoreMesh(core_axis_name="c", subcore_axis_name="s")
    rows_per_w = n_rows // NW
    nc = rows_per_w // chunk

    @functools.partial(
        pl.kernel,
        out_type=jax.ShapeDtypeStruct((n_rows, d_model), dtype),
        mesh=mesh,
        scratch_types=[
            pltpu.VMEM((rows_per_w,), jnp.int32),
            pltpu.VMEM((2, chunk, d_model), dtype),
            pltpu.SemaphoreType.DMA,
            pltpu.SemaphoreType.DMA,
            pltpu.SemaphoreType.DMA,
            pltpu.SemaphoreType.DMA,
        ],
    )
    def gather_k(x_hbm, rows_hbm, out_hbm, idx_v, buf_v, g0, g1, s0, s1):
        wid = lax.axis_index("s") * 2 + lax.axis_index("c")
        base = wid * rows_per_w
        gsem = (g0, g1)
        ssem = (s0, s1)
        pltpu.sync_copy(rows_hbm.at[pl.ds(base, rows_per_w)], idx_v)

        gh = [None] * nc
        sh = [None] * nc
        gh[0] = pltpu.async_copy(
            x_hbm.at[idx_v.at[pl.ds(0, chunk)]], buf_v.at[0], gsem[0]
        )
        for c in range(nc):
            if c + 1 < nc:
                b = (c + 1) % 2
                if c - 1 >= 0:
                    sh[c - 1].wait()
                gh[c + 1] = pltpu.async_copy(
                    x_hbm.at[idx_v.at[pl.ds((c + 1) * chunk, chunk)]],
                    buf_v.at[b],
                    gsem[b],
                )
            gh[c].wait()
            sh[c] = pltpu.async_copy(
                buf_v.at[c % 2], out_hbm.at[pl.ds(base + c * chunk, chunk)],
                ssem[c % 2],
            )
        sh[nc - 2].wait()
        sh[nc - 1].wait()

    return gather_k


def _make_combine(n_tokens, d_model, chunk):
    mesh = plsc.VectorSubcoreMesh(core_axis_name="c", subcore_axis_name="s")
    tok_per_w = n_tokens // NW
    nc = tok_per_w // chunk
    lanes_per_row = d_model // 16

    @functools.partial(
        pl.kernel,
        out_type=jax.ShapeDtypeStruct((n_tokens, d_model), jnp.float32),
        mesh=mesh,
        scratch_types=[
            pltpu.VMEM((tok_per_w,), jnp.int32),
            pltpu.VMEM((tok_per_w,), jnp.int32),
            pltpu.VMEM((2, chunk, d_model), jnp.float32),
            pltpu.VMEM((2, chunk, d_model), jnp.float32),
            pltpu.SemaphoreType.DMA,
            pltpu.SemaphoreType.DMA,
            pltpu.SemaphoreType.DMA,
            pltpu.SemaphoreType.DMA,
            pltpu.SemaphoreType.DMA,
            pltpu.SemaphoreType.DMA,
        ],
    )
    def combine_k(y_hbm, r0_hbm, r1_hbm, out_hbm, i0_v, i1_v, a_v, b_v,
                  ga0, ga1, gb0, gb1, ss0, ss1):
        wid = lax.axis_index("s") * 2 + lax.axis_index("c")
        base = wid * tok_per_w
        ga = (ga0, ga1)
        gb = (gb0, gb1)
        ss = (ss0, ss1)
        pltpu.sync_copy(r0_hbm.at[pl.ds(base, tok_per_w)], i0_v)
        pltpu.sync_copy(r1_hbm.at[pl.ds(base, tok_per_w)], i1_v)

        def start(c, b):
            ha = pltpu.async_copy(
                y_hbm.at[i0_v.at[pl.ds(c * chunk, chunk)]], a_v.at[b], ga[b]
            )
            hb = pltpu.async_copy(
                y_hbm.at[i1_v.at[pl.ds(c * chunk, chunk)]], b_v.at[b], gb[b]
            )
            return ha, hb

        hs = [None] * nc
        st = [None] * nc
        hs[0] = start(0, 0)
        for c in range(nc):
            b = c % 2
            if c + 1 < nc:
                if c - 1 >= 0:
                    st[c - 1].wait()
                hs[c + 1] = start(c + 1, (c + 1) % 2)
            hs[c][0].wait()
            hs[c][1].wait()

            def row_add(r, carry):
                def col_add(k, c3):
                    for u in range(8):
                        sl = pl.ds((k * 8 + u) * 16, 16)
                        a_v[b, r, sl] = a_v[b, r, sl] + b_v[b, r, sl]
                    return c3

                lax.fori_loop(0, lanes_per_row // 8, col_add, 0)
                return carry

            lax.fori_loop(0, chunk, row_add, 0)
            st[c] = pltpu.async_copy(
                a_v.at[b], out_hbm.at[pl.ds(base + c * chunk, chunk)], ss[b]
            )
        if nc >= 2:
            st[nc - 2].wait()
        st[nc - 1].wait()

    return combine_k


def _gmm_body(te_ref, mi_ref, valid_ref, xg_ref, wg_ref, wu_ref, wd_ref,
              wrow_ref, y_ref):
    i = pl.program_id(0)
    j = pl.program_id(1)

    @pl.when(valid_ref[i] != 0)
    def _():
        x = xg_ref[...]
        g = jnp.dot(x, wg_ref[0].T, preferred_element_type=jnp.float32)
        u = jnp.dot(x, wu_ref[0].T, preferred_element_type=jnp.float32)
        h = (g * jax.nn.sigmoid(g)) * u
        yj = jnp.dot(h, wd_ref[0].T, preferred_element_type=jnp.float32)
        yj = yj * wrow_ref[0, 0, :][:, None]

        @pl.when(j == 0)
        def _():
            y_ref[...] = jnp.zeros_like(y_ref)

        y_ref[...] += yj


def _make_gmm(n_rows, d_model, d_expert, n_experts, blk_m, blk_n):
    nt = n_rows // blk_m
    nb = d_expert // blk_n
    grid_spec = pltpu.PrefetchScalarGridSpec(
        num_scalar_prefetch=3,
        grid=(nt, nb),
        in_specs=[
            pl.BlockSpec((blk_m, d_model), lambda i, j, te, mi, v: (mi[i], 0)),
            pl.BlockSpec((1, blk_n, d_model), lambda i, j, te, mi, v: (te[i], j, 0)),
            pl.BlockSpec((1, blk_n, d_model), lambda i, j, te, mi, v: (te[i], j, 0)),
            pl.BlockSpec((1, d_model, blk_n), lambda i, j, te, mi, v: (te[i], 0, j)),
            pl.BlockSpec((1, 1, blk_m), lambda i, j, te, mi, v: (mi[i], 0, 0)),
        ],
        out_specs=pl.BlockSpec((blk_m, d_model), lambda i, j, te, mi, v: (mi[i], 0)),
    )
    return pl.pallas_call(
        _gmm_body,
        grid_spec=grid_spec,
        out_shape=jax.ShapeDtypeStruct((n_rows, d_model), jnp.float32),
        compiler_params=pltpu.CompilerParams(
            dimension_semantics=("arbitrary", "arbitrary"),
        ),
    )


def kernel(x, routing_weights, expert_indices, w_gate, w_up, w_down):
    batch, seq_len, d_model = x.shape
    top_k = expert_indices.shape[-1]
    n_experts, d_expert, _ = w_gate.shape
    n_tokens = batch * seq_len
    n_assign = n_tokens * top_k

    blk_m = 512
    blk_n = 512
    n_rows = n_assign + n_experts * blk_m

    x_flat = x.reshape(n_tokens, d_model)
    e_flat = expert_indices.reshape(n_assign).astype(jnp.int32)
    w_flat = routing_weights.reshape(n_assign).astype(jnp.float32)

    order = jnp.argsort(e_flat)
    e_sorted = jnp.take(e_flat, order)
    counts = jnp.bincount(e_flat, length=n_experts)
    starts = jnp.cumsum(counts) - counts
    pc = ((counts + blk_m - 1) // blk_m) * blk_m
    padded_starts = jnp.cumsum(pc) - pc
    p = jnp.arange(n_assign, dtype=jnp.int32)
    row_sorted = (padded_starts[e_sorted] + (p - starts[e_sorted])).astype(jnp.int32)
    token_for_sorted = (order // top_k).astype(jnp.int32)
    token_row = jnp.zeros((n_rows,), jnp.int32).at[row_sorted].set(token_for_sorted)
    w_row = jnp.zeros((n_rows,), jnp.float32).at[row_sorted].set(jnp.take(w_flat, order))
    nt = n_rows // blk_m
    tile_start = jnp.arange(nt, dtype=jnp.int32) * blk_m
    pcum = jnp.cumsum(pc)
    total_rows = pcum[-1]
    tile_valid = (tile_start < total_rows).astype(jnp.int32)
    n_valid = total_rows // blk_m
    te_raw = jnp.minimum(
        jnp.searchsorted(pcum, tile_start, side="right"), n_experts - 1
    ).astype(jnp.int32)
    last_te = jnp.take(te_raw, n_valid - 1)
    tile_expert = jnp.where(tile_valid == 1, te_raw, last_te)
    tile_mi = jnp.minimum(
        jnp.arange(nt, dtype=jnp.int32), (n_valid - 1).astype(jnp.int32)
    )
    row_by_a = jnp.zeros((n_assign,), jnp.int32).at[order].set(row_sorted)
    r0 = row_by_a[0::top_k]
    r1 = row_by_a[1::top_k]

    xg = _make_gather(n_rows, d_model, 48, jnp.float32)(x_flat, token_row)

    w_row3 = w_row.reshape(nt, 1, blk_m)
    y = _make_gmm(n_rows, d_model, d_expert, n_experts, blk_m, blk_n)(
        tile_expert, tile_mi, tile_valid, xg, w_gate, w_up, w_down, w_row3
    )

    out = _make_combine(n_tokens, d_model, chunk=16)(y, r0, r1)
    return out.reshape(batch, seq_len, d_model)


# speedup vs baseline: 2.0564x; 1.2994x over previous
import functools

import jax
import jax.numpy as jnp
from jax import lax
from jax.experimental import pallas as pl
from jax.experimental.pallas import tpu as pltpu
from jax.experimental.pallas import tpu_sc as plsc

NW = 32


def _make_dispatch(n_tokens, n_rows, d_model, chunk):
    mesh = plsc.VectorSubcoreMesh(core_axis_name="c", subcore_axis_name="s")
    tok_per_w = n_tokens // NW
    nc = tok_per_w // chunk

    @functools.partial(
        pl.kernel,
        out_type=jax.ShapeDtypeStruct((n_rows, d_model), jnp.float32),
        mesh=mesh,
        scratch_types=[
            pltpu.VMEM((nc, chunk), jnp.int32),
            pltpu.VMEM((nc, chunk), jnp.int32),
            pltpu.VMEM((2, chunk, d_model), jnp.float32),
            pltpu.SemaphoreType.DMA,
            pltpu.SemaphoreType.DMA,
            pltpu.SemaphoreType.DMA,
            pltpu.SemaphoreType.DMA,
            pltpu.SemaphoreType.DMA,
            pltpu.SemaphoreType.DMA,
        ],
    )
    def dispatch_k(x_hbm, r0_hbm, r1_hbm, out_hbm, i0_v, i1_v, buf_v,
                   l0, l1, sa0, sa1, sb0, sb1):
        wid = lax.axis_index("s") * 2 + lax.axis_index("c")
        base = wid * tok_per_w
        lin = (l0, l1)
        sa = (sa0, sa1)
        sb = (sb0, sb1)
        pltpu.sync_copy(r0_hbm.at[wid], i0_v)
        pltpu.sync_copy(r1_hbm.at[wid], i1_v)

        lh = [None] * nc
        sha = [None] * nc
        shb = [None] * nc
        lh[0] = pltpu.async_copy(
            x_hbm.at[pl.ds(base, chunk)], buf_v.at[0], lin[0]
        )
        for c in range(nc):
            b = c % 2
            if c + 1 < nc:
                b2 = (c + 1) % 2
                if c - 1 >= 0:
                    sha[c - 1].wait()
                    shb[c - 1].wait()
                lh[c + 1] = pltpu.async_copy(
                    x_hbm.at[pl.ds(base + (c + 1) * chunk, chunk)],
                    buf_v.at[b2],
                    lin[b2],
                )
            lh[c].wait()
            sha[c] = pltpu.async_copy(buf_v.at[b], out_hbm.at[i0_v.at[c]], sa[b])
            shb[c] = pltpu.async_copy(buf_v.at[b], out_hbm.at[i1_v.at[c]], sb[b])
        if nc >= 2:
            sha[nc - 2].wait()
            shb[nc - 2].wait()
        sha[nc - 1].wait()
        shb[nc - 1].wait()

    return dispatch_k


def _make_combine(n_tokens, d_model, chunk):
    mesh = plsc.VectorSubcoreMesh(core_axis_name="c", subcore_axis_name="s")
    tok_per_w = n_tokens // NW
    nc = tok_per_w // chunk
    lanes_per_row = d_model // 16

    @functools.partial(
        pl.kernel,
        out_type=jax.ShapeDtypeStruct((n_tokens, d_model), jnp.float32),
        mesh=mesh,
        scratch_types=[
            pltpu.VMEM((tok_per_w,), jnp.int32),
            pltpu.VMEM((tok_per_w,), jnp.int32),
            pltpu.VMEM((2, chunk, d_model), jnp.float32),
            pltpu.VMEM((2, chunk, d_model), jnp.float32),
            pltpu.SemaphoreType.DMA,
            pltpu.SemaphoreType.DMA,
            pltpu.SemaphoreType.DMA,
            pltpu.SemaphoreType.DMA,
            pltpu.SemaphoreType.DMA,
            pltpu.SemaphoreType.DMA,
        ],
    )
    def combine_k(y_hbm, r0_hbm, r1_hbm, out_hbm, i0_v, i1_v, a_v, b_v,
                  ga0, ga1, gb0, gb1, ss0, ss1):
        wid = lax.axis_index("s") * 2 + lax.axis_index("c")
        base = wid * tok_per_w
        ga = (ga0, ga1)
        gb = (gb0, gb1)
        ss = (ss0, ss1)
        pltpu.sync_copy(r0_hbm.at[pl.ds(base, tok_per_w)], i0_v)
        pltpu.sync_copy(r1_hbm.at[pl.ds(base, tok_per_w)], i1_v)

        def start(c, b):
            ha = pltpu.async_copy(
                y_hbm.at[i0_v.at[pl.ds(c * chunk, chunk)]], a_v.at[b], ga[b]
            )
            hb = pltpu.async_copy(
                y_hbm.at[i1_v.at[pl.ds(c * chunk, chunk)]], b_v.at[b], gb[b]
            )
            return ha, hb

        hs = [None] * nc
        st = [None] * nc
        hs[0] = start(0, 0)
        for c in range(nc):
            b = c % 2
            if c + 1 < nc:
                if c - 1 >= 0:
                    st[c - 1].wait()
                hs[c + 1] = start(c + 1, (c + 1) % 2)
            hs[c][0].wait()
            hs[c][1].wait()

            def row_add(r, carry):
                def col_add(k, c3):
                    for u in range(8):
                        sl = pl.ds((k * 8 + u) * 16, 16)
                        a_v[b, r, sl] = a_v[b, r, sl] + b_v[b, r, sl]
                    return c3

                lax.fori_loop(0, lanes_per_row // 8, col_add, 0)
                return carry

            lax.fori_loop(0, chunk, row_add, 0)
            st[c] = pltpu.async_copy(
                a_v.at[b], out_hbm.at[pl.ds(base + c * chunk, chunk)], ss[b]
            )
        if nc >= 2:
            st[nc - 2].wait()
        st[nc - 1].wait()

    return combine_k


def _gmm_body(te_ref, mi_ref, valid_ref, xg_ref, wg_ref, wu_ref, wd_ref,
              wrow_ref, y_ref):
    i = pl.program_id(0)
    j = pl.program_id(1)

    @pl.when(valid_ref[i] != 0)
    def _():
        x = xg_ref[...]
        g = jnp.dot(x, wg_ref[0].T, preferred_element_type=jnp.float32)
        u = jnp.dot(x, wu_ref[0].T, preferred_element_type=jnp.float32)
        h = (g * jax.nn.sigmoid(g)) * u
        yj = jnp.dot(h, wd_ref[0].T, preferred_element_type=jnp.float32)
        yj = yj * wrow_ref[0, 0, :][:, None]

        @pl.when(j == 0)
        def _():
            y_ref[...] = jnp.zeros_like(y_ref)

        y_ref[...] += yj


def _make_gmm(n_rows, d_model, d_expert, n_experts, blk_m, blk_n):
    nt = n_rows // blk_m
    nb = d_expert // blk_n
    grid_spec = pltpu.PrefetchScalarGridSpec(
        num_scalar_prefetch=3,
        grid=(nt, nb),
        in_specs=[
            pl.BlockSpec((blk_m, d_model), lambda i, j, te, mi, v: (mi[i], 0)),
            pl.BlockSpec((1, blk_n, d_model), lambda i, j, te, mi, v: (te[i], j, 0)),
            pl.BlockSpec((1, blk_n, d_model), lambda i, j, te, mi, v: (te[i], j, 0)),
            pl.BlockSpec((1, d_model, blk_n), lambda i, j, te, mi, v: (te[i], 0, j)),
            pl.BlockSpec((1, 1, blk_m), lambda i, j, te, mi, v: (mi[i], 0, 0)),
        ],
        out_specs=pl.BlockSpec((blk_m, d_model), lambda i, j, te, mi, v: (mi[i], 0)),
    )
    return pl.pallas_call(
        _gmm_body,
        grid_spec=grid_spec,
        out_shape=jax.ShapeDtypeStruct((n_rows, d_model), jnp.float32),
        compiler_params=pltpu.CompilerParams(
            dimension_semantics=("arbitrary", "arbitrary"),
        ),
    )


def kernel(x, routing_weights, expert_indices, w_gate, w_up, w_down):
    batch, seq_len, d_model = x.shape
    top_k = expert_indices.shape[-1]
    n_experts, d_expert, _ = w_gate.shape
    n_tokens = batch * seq_len
    n_assign = n_tokens * top_k

    blk_m = 512
    blk_n = 512
    n_rows = n_assign + n_experts * blk_m

    x_flat = x.reshape(n_tokens, d_model)
    e_flat = expert_indices.reshape(n_assign).astype(jnp.int32)
    w_flat = routing_weights.reshape(n_assign).astype(jnp.float32)

    order = jnp.argsort(e_flat)
    e_sorted = jnp.take(e_flat, order)
    counts = jnp.bincount(e_flat, length=n_experts)
    starts = jnp.cumsum(counts) - counts
    pc = ((counts + blk_m - 1) // blk_m) * blk_m
    padded_starts = jnp.cumsum(pc) - pc
    p = jnp.arange(n_assign, dtype=jnp.int32)
    row_sorted = (padded_starts[e_sorted] + (p - starts[e_sorted])).astype(jnp.int32)
    w_row = jnp.zeros((n_rows,), jnp.float32).at[row_sorted].set(jnp.take(w_flat, order))
    nt = n_rows // blk_m
    tile_start = jnp.arange(nt, dtype=jnp.int32) * blk_m
    pcum = jnp.cumsum(pc)
    total_rows = pcum[-1]
    tile_valid = (tile_start < total_rows).astype(jnp.int32)
    n_valid = total_rows // blk_m
    te_raw = jnp.minimum(
        jnp.searchsorted(pcum, tile_start, side="right"), n_experts - 1
    ).astype(jnp.int32)
    last_te = jnp.take(te_raw, n_valid - 1)
    tile_expert = jnp.where(tile_valid == 1, te_raw, last_te)
    tile_mi = jnp.minimum(
        jnp.arange(nt, dtype=jnp.int32), (n_valid - 1).astype(jnp.int32)
    )
    row_by_a = jnp.zeros((n_assign,), jnp.int32).at[order].set(row_sorted)
    r0 = row_by_a[0::top_k]
    r1 = row_by_a[1::top_k]

    d_chunk = 32
    d_nc = n_tokens // NW // d_chunk
    r0_3 = r0.reshape(NW, d_nc, d_chunk)
    r1_3 = r1.reshape(NW, d_nc, d_chunk)
    xg = _make_dispatch(n_tokens, n_rows, d_model, d_chunk)(x_flat, r0_3, r1_3)

    w_row3 = w_row.reshape(nt, 1, blk_m)
    y = _make_gmm(n_rows, d_model, d_expert, n_experts, blk_m, blk_n)(
        tile_expert, tile_mi, tile_valid, xg, w_gate, w_up, w_down, w_row3
    )

    out = _make_combine(n_tokens, d_model, chunk=16)(y, r0, r1)
    return out.reshape(batch, seq_len, d_model)


# speedup vs baseline: 2.2615x; 1.0998x over previous
import functools

import jax
import jax.numpy as jnp
from jax import lax
from jax.experimental import pallas as pl
from jax.experimental.pallas import tpu as pltpu
from jax.experimental.pallas import tpu_sc as plsc

NW = 32


def _make_dispatch(n_tokens, n_rows, d_model, chunk):
    mesh = plsc.VectorSubcoreMesh(core_axis_name="c", subcore_axis_name="s")
    tok_per_w = n_tokens // NW
    nc = tok_per_w // chunk

    @functools.partial(
        pl.kernel,
        out_type=jax.ShapeDtypeStruct((n_rows, d_model), jnp.float32),
        mesh=mesh,
        scratch_types=[
            pltpu.VMEM((nc, chunk), jnp.int32),
            pltpu.VMEM((nc, chunk), jnp.int32),
            pltpu.VMEM((2, chunk, d_model), jnp.float32),
            pltpu.SemaphoreType.DMA,
            pltpu.SemaphoreType.DMA,
            pltpu.SemaphoreType.DMA,
            pltpu.SemaphoreType.DMA,
            pltpu.SemaphoreType.DMA,
            pltpu.SemaphoreType.DMA,
        ],
    )
    def dispatch_k(x_hbm, r0_hbm, r1_hbm, out_hbm, i0_v, i1_v, buf_v,
                   l0, l1, sa0, sa1, sb0, sb1):
        wid = lax.axis_index("s") * 2 + lax.axis_index("c")
        base = wid * tok_per_w
        lin = (l0, l1)
        sa = (sa0, sa1)
        sb = (sb0, sb1)
        pltpu.sync_copy(r0_hbm.at[wid], i0_v)
        pltpu.sync_copy(r1_hbm.at[wid], i1_v)

        lh = [None] * nc
        sha = [None] * nc
        shb = [None] * nc
        lh[0] = pltpu.async_copy(
            x_hbm.at[pl.ds(base, chunk)], buf_v.at[0], lin[0]
        )
        for c in range(nc):
            b = c % 2
            if c + 1 < nc:
                b2 = (c + 1) % 2
                if c - 1 >= 0:
                    sha[c - 1].wait()
                    shb[c - 1].wait()
                lh[c + 1] = pltpu.async_copy(
                    x_hbm.at[pl.ds(base + (c + 1) * chunk, chunk)],
                    buf_v.at[b2],
                    lin[b2],
                )
            lh[c].wait()
            sha[c] = pltpu.async_copy(buf_v.at[b], out_hbm.at[i0_v.at[c]], sa[b])
            shb[c] = pltpu.async_copy(buf_v.at[b], out_hbm.at[i1_v.at[c]], sb[b])
        if nc >= 2:
            sha[nc - 2].wait()
            shb[nc - 2].wait()
        sha[nc - 1].wait()
        shb[nc - 1].wait()

    return dispatch_k


def _make_combine(n_tokens, d_model, chunk):
    mesh = plsc.VectorSubcoreMesh(core_axis_name="c", subcore_axis_name="s")
    tok_per_w = n_tokens // NW
    nc = tok_per_w // chunk
    lanes_per_row = d_model // 16

    @functools.partial(
        pl.kernel,
        out_type=jax.ShapeDtypeStruct((n_tokens, d_model), jnp.float32),
        mesh=mesh,
        scratch_types=[
            pltpu.VMEM((tok_per_w,), jnp.int32),
            pltpu.VMEM((tok_per_w,), jnp.int32),
            pltpu.VMEM((2, chunk, d_model), jnp.float32),
            pltpu.VMEM((2, chunk, d_model), jnp.float32),
            pltpu.SemaphoreType.DMA,
            pltpu.SemaphoreType.DMA,
            pltpu.SemaphoreType.DMA,
            pltpu.SemaphoreType.DMA,
            pltpu.SemaphoreType.DMA,
            pltpu.SemaphoreType.DMA,
        ],
    )
    def combine_k(y_hbm, r0_hbm, r1_hbm, out_hbm, i0_v, i1_v, a_v, b_v,
                  ga0, ga1, gb0, gb1, ss0, ss1):
        wid = lax.axis_index("s") * 2 + lax.axis_index("c")
        base = wid * tok_per_w
        ga = (ga0, ga1)
        gb = (gb0, gb1)
        ss = (ss0, ss1)
        pltpu.sync_copy(r0_hbm.at[pl.ds(base, tok_per_w)], i0_v)
        pltpu.sync_copy(r1_hbm.at[pl.ds(base, tok_per_w)], i1_v)

        def start(c, b):
            ha = pltpu.async_copy(
                y_hbm.at[i0_v.at[pl.ds(c * chunk, chunk)]], a_v.at[b], ga[b]
            )
            hb = pltpu.async_copy(
                y_hbm.at[i1_v.at[pl.ds(c * chunk, chunk)]], b_v.at[b], gb[b]
            )
            return ha, hb

        hs = [None] * nc
        st = [None] * nc
        hs[0] = start(0, 0)
        for c in range(nc):
            b = c % 2
            if c + 1 < nc:
                if c - 1 >= 0:
                    st[c - 1].wait()
                hs[c + 1] = start(c + 1, (c + 1) % 2)
            hs[c][0].wait()
            hs[c][1].wait()

            def row_add(r, carry):
                def col_add(k, c3):
                    for u in range(8):
                        sl = pl.ds((k * 8 + u) * 16, 16)
                        a_v[b, r, sl] = a_v[b, r, sl] + b_v[b, r, sl]
                    return c3

                lax.fori_loop(0, lanes_per_row // 8, col_add, 0)
                return carry

            lax.fori_loop(0, chunk, row_add, 0)
            st[c] = pltpu.async_copy(
                a_v.at[b], out_hbm.at[pl.ds(base + c * chunk, chunk)], ss[b]
            )
        if nc >= 2:
            st[nc - 2].wait()
        st[nc - 1].wait()

    return combine_k


def _gmm_body(te_ref, mi_ref, valid_ref, xg_ref, wg_ref, wu_ref, wd_ref,
              wrow_ref, y_ref):
    i = pl.program_id(0)
    j = pl.program_id(1)

    @pl.when(valid_ref[i] != 0)
    def _():
        x = xg_ref[...]
        g = jnp.dot(x, wg_ref[0].T, preferred_element_type=jnp.float32)
        u = jnp.dot(x, wu_ref[0].T, preferred_element_type=jnp.float32)
        h = (g * jax.nn.sigmoid(g)) * u
        yj = jnp.dot(h, wd_ref[0].T, preferred_element_type=jnp.float32)
        yj = yj * wrow_ref[0, 0, :][:, None]

        @pl.when(j == 0)
        def _():
            y_ref[...] = jnp.zeros_like(y_ref)

        y_ref[...] += yj


def _make_gmm(n_rows, d_model, d_expert, n_experts, blk_m, blk_n):
    nt = n_rows // blk_m
    nb = d_expert // blk_n
    grid_spec = pltpu.PrefetchScalarGridSpec(
        num_scalar_prefetch=3,
        grid=(nt, nb),
        in_specs=[
            pl.BlockSpec((blk_m, d_model), lambda i, j, te, mi, v: (mi[i], 0)),
            pl.BlockSpec((1, blk_n, d_model), lambda i, j, te, mi, v: (te[i], j, 0)),
            pl.BlockSpec((1, blk_n, d_model), lambda i, j, te, mi, v: (te[i], j, 0)),
            pl.BlockSpec((1, d_model, blk_n), lambda i, j, te, mi, v: (te[i], 0, j)),
            pl.BlockSpec((1, 1, blk_m), lambda i, j, te, mi, v: (mi[i], 0, 0)),
        ],
        out_specs=pl.BlockSpec((blk_m, d_model), lambda i, j, te, mi, v: (mi[i], 0)),
    )
    return pl.pallas_call(
        _gmm_body,
        grid_spec=grid_spec,
        out_shape=jax.ShapeDtypeStruct((n_rows, d_model), jnp.float32),
        compiler_params=pltpu.CompilerParams(
            dimension_semantics=("arbitrary", "arbitrary"),
        ),
    )


def kernel(x, routing_weights, expert_indices, w_gate, w_up, w_down):
    batch, seq_len, d_model = x.shape
    top_k = expert_indices.shape[-1]
    n_experts, d_expert, _ = w_gate.shape
    n_tokens = batch * seq_len
    n_assign = n_tokens * top_k

    blk_m = 512
    blk_n = 1024
    n_rows = n_assign + n_experts * blk_m

    x_flat = x.reshape(n_tokens, d_model)
    e_flat = expert_indices.reshape(n_assign).astype(jnp.int32)
    w_flat = routing_weights.reshape(n_assign).astype(jnp.float32)

    order = jnp.argsort(e_flat)
    e_sorted = jnp.take(e_flat, order)
    counts = jnp.bincount(e_flat, length=n_experts)
    starts = jnp.cumsum(counts) - counts
    pc = ((counts + blk_m - 1) // blk_m) * blk_m
    padded_starts = jnp.cumsum(pc) - pc
    p = jnp.arange(n_assign, dtype=jnp.int32)
    row_sorted = (padded_starts[e_sorted] + (p - starts[e_sorted])).astype(jnp.int32)
    w_row = jnp.zeros((n_rows,), jnp.float32).at[row_sorted].set(jnp.take(w_flat, order))
    nt = n_rows // blk_m
    tile_start = jnp.arange(nt, dtype=jnp.int32) * blk_m
    pcum = jnp.cumsum(pc)
    total_rows = pcum[-1]
    tile_valid = (tile_start < total_rows).astype(jnp.int32)
    n_valid = total_rows // blk_m
    te_raw = jnp.minimum(
        jnp.searchsorted(pcum, tile_start, side="right"), n_experts - 1
    ).astype(jnp.int32)
    last_te = jnp.take(te_raw, n_valid - 1)
    tile_expert = jnp.where(tile_valid == 1, te_raw, last_te)
    tile_mi = jnp.minimum(
        jnp.arange(nt, dtype=jnp.int32), (n_valid - 1).astype(jnp.int32)
    )
    row_by_a = jnp.zeros((n_assign,), jnp.int32).at[order].set(row_sorted)
    r0 = row_by_a[0::top_k]
    r1 = row_by_a[1::top_k]

    d_chunk = 32
    d_nc = n_tokens // NW // d_chunk
    r0_3 = r0.reshape(NW, d_nc, d_chunk)
    r1_3 = r1.reshape(NW, d_nc, d_chunk)
    xg = _make_dispatch(n_tokens, n_rows, d_model, d_chunk)(x_flat, r0_3, r1_3)

    w_row3 = w_row.reshape(nt, 1, blk_m)
    y = _make_gmm(n_rows, d_model, d_expert, n_experts, blk_m, blk_n)(
        tile_expert, tile_mi, tile_valid, xg, w_gate, w_up, w_down, w_row3
    )

    out = _make_combine(n_tokens, d_model, chunk=16)(y, r0, r1)
    return out.reshape(batch, seq_len, d_model)


# speedup vs baseline: 2.2814x; 1.0088x over previous
import functools

import jax
import jax.numpy as jnp
from jax import lax
from jax.experimental import pallas as pl
from jax.experimental.pallas import tpu as pltpu
from jax.experimental.pallas import tpu_sc as plsc

NW = 32


def _make_dispatch(n_tokens, n_rows, d_model, chunk):
    mesh = plsc.VectorSubcoreMesh(core_axis_name="c", subcore_axis_name="s")
    tok_per_w = n_tokens // NW
    nc = tok_per_w // chunk

    @functools.partial(
        pl.kernel,
        out_type=jax.ShapeDtypeStruct((n_rows, d_model), jnp.float32),
        mesh=mesh,
        scratch_types=[
            pltpu.VMEM((nc, chunk), jnp.int32),
            pltpu.VMEM((nc, chunk), jnp.int32),
            pltpu.VMEM((2, chunk, d_model), jnp.float32),
            pltpu.SemaphoreType.DMA,
            pltpu.SemaphoreType.DMA,
            pltpu.SemaphoreType.DMA,
            pltpu.SemaphoreType.DMA,
            pltpu.SemaphoreType.DMA,
            pltpu.SemaphoreType.DMA,
        ],
    )
    def dispatch_k(x_hbm, r0_hbm, r1_hbm, out_hbm, i0_v, i1_v, buf_v,
                   l0, l1, sa0, sa1, sb0, sb1):
        wid = lax.axis_index("s") * 2 + lax.axis_index("c")
        base = wid * tok_per_w
        lin = (l0, l1)
        sa = (sa0, sa1)
        sb = (sb0, sb1)
        pltpu.sync_copy(r0_hbm.at[wid], i0_v)
        pltpu.sync_copy(r1_hbm.at[wid], i1_v)

        lh = [None] * nc
        sha = [None] * nc
        shb = [None] * nc
        lh[0] = pltpu.async_copy(
            x_hbm.at[pl.ds(base, chunk)], buf_v.at[0], lin[0]
        )
        for c in range(nc):
            b = c % 2
            if c + 1 < nc:
                b2 = (c + 1) % 2
                if c - 1 >= 0:
                    sha[c - 1].wait()
                    shb[c - 1].wait()
                lh[c + 1] = pltpu.async_copy(
                    x_hbm.at[pl.ds(base + (c + 1) * chunk, chunk)],
                    buf_v.at[b2],
                    lin[b2],
                )
            lh[c].wait()
            sha[c] = pltpu.async_copy(buf_v.at[b], out_hbm.at[i0_v.at[c]], sa[b])
            shb[c] = pltpu.async_copy(buf_v.at[b], out_hbm.at[i1_v.at[c]], sb[b])
        if nc >= 2:
            sha[nc - 2].wait()
            shb[nc - 2].wait()
        sha[nc - 1].wait()
        shb[nc - 1].wait()

    return dispatch_k


def _make_combine(n_tokens, d_model, chunk):
    mesh = plsc.VectorSubcoreMesh(core_axis_name="c", subcore_axis_name="s")
    tok_per_w = n_tokens // NW
    nc = tok_per_w // chunk
    lanes_per_row = d_model // 16

    @functools.partial(
        pl.kernel,
        out_type=jax.ShapeDtypeStruct((n_tokens, d_model), jnp.float32),
        mesh=mesh,
        scratch_types=[
            pltpu.VMEM((tok_per_w,), jnp.int32),
            pltpu.VMEM((tok_per_w,), jnp.int32),
            pltpu.VMEM((2, chunk, d_model), jnp.float32),
            pltpu.VMEM((2, chunk, d_model), jnp.float32),
            pltpu.SemaphoreType.DMA,
            pltpu.SemaphoreType.DMA,
            pltpu.SemaphoreType.DMA,
            pltpu.SemaphoreType.DMA,
            pltpu.SemaphoreType.DMA,
            pltpu.SemaphoreType.DMA,
        ],
    )
    def combine_k(y_hbm, r0_hbm, r1_hbm, out_hbm, i0_v, i1_v, a_v, b_v,
                  ga0, ga1, gb0, gb1, ss0, ss1):
        wid = lax.axis_index("s") * 2 + lax.axis_index("c")
        base = wid * tok_per_w
        ga = (ga0, ga1)
        gb = (gb0, gb1)
        ss = (ss0, ss1)
        pltpu.sync_copy(r0_hbm.at[pl.ds(base, tok_per_w)], i0_v)
        pltpu.sync_copy(r1_hbm.at[pl.ds(base, tok_per_w)], i1_v)

        def start(c, b):
            ha = pltpu.async_copy(
                y_hbm.at[i0_v.at[pl.ds(c * chunk, chunk)]], a_v.at[b], ga[b]
            )
            hb = pltpu.async_copy(
                y_hbm.at[i1_v.at[pl.ds(c * chunk, chunk)]], b_v.at[b], gb[b]
            )
            return ha, hb

        hs = [None] * nc
        st = [None] * nc
        hs[0] = start(0, 0)
        for c in range(nc):
            b = c % 2
            if c + 1 < nc:
                if c - 1 >= 0:
                    st[c - 1].wait()
                hs[c + 1] = start(c + 1, (c + 1) % 2)
            hs[c][0].wait()
            hs[c][1].wait()

            def row_add(r, carry):
                def col_add(k, c3):
                    for u in range(8):
                        sl = pl.ds((k * 8 + u) * 16, 16)
                        a_v[b, r, sl] = a_v[b, r, sl] + b_v[b, r, sl]
                    return c3

                lax.fori_loop(0, lanes_per_row // 8, col_add, 0)
                return carry

            lax.fori_loop(0, chunk, row_add, 0)
            st[c] = pltpu.async_copy(
                a_v.at[b], out_hbm.at[pl.ds(base + c * chunk, chunk)], ss[b]
            )
        if nc >= 2:
            st[nc - 2].wait()
        st[nc - 1].wait()

    return combine_k


def _gmm_body(te_ref, mi_ref, valid_ref, xg_ref, wg_ref, wu_ref, wd_ref,
              wrow_ref, y_ref):
    i = pl.program_id(0)
    j = pl.program_id(1)

    @pl.when(valid_ref[i] != 0)
    def _():
        x = xg_ref[...]
        g = jnp.dot(x, wg_ref[0].T, preferred_element_type=jnp.float32)
        u = jnp.dot(x, wu_ref[0].T, preferred_element_type=jnp.float32)
        h = (g * jax.nn.sigmoid(g)) * u
        yj = jnp.dot(h, wd_ref[0].T, preferred_element_type=jnp.float32)
        yj = yj * wrow_ref[0, 0, :][:, None]

        @pl.when(j == 0)
        def _():
            y_ref[...] = jnp.zeros_like(y_ref)

        y_ref[...] += yj


def _make_gmm(n_rows, d_model, d_expert, n_experts, blk_m, blk_n):
    nt = n_rows // blk_m
    nb = d_expert // blk_n
    grid_spec = pltpu.PrefetchScalarGridSpec(
        num_scalar_prefetch=3,
        grid=(nt, nb),
        in_specs=[
            pl.BlockSpec((blk_m, d_model), lambda i, j, te, mi, v: (mi[i], 0)),
            pl.BlockSpec((1, blk_n, d_model), lambda i, j, te, mi, v: (te[i], j, 0)),
            pl.BlockSpec((1, blk_n, d_model), lambda i, j, te, mi, v: (te[i], j, 0)),
            pl.BlockSpec((1, d_model, blk_n), lambda i, j, te, mi, v: (te[i], 0, j)),
            pl.BlockSpec((1, 1, blk_m), lambda i, j, te, mi, v: (mi[i], 0, 0)),
        ],
        out_specs=pl.BlockSpec((blk_m, d_model), lambda i, j, te, mi, v: (mi[i], 0)),
    )
    return pl.pallas_call(
        _gmm_body,
        grid_spec=grid_spec,
        out_shape=jax.ShapeDtypeStruct((n_rows, d_model), jnp.float32),
        compiler_params=pltpu.CompilerParams(
            dimension_semantics=("arbitrary", "arbitrary"),
        ),
    )


def kernel(x, routing_weights, expert_indices, w_gate, w_up, w_down):
    batch, seq_len, d_model = x.shape
    top_k = expert_indices.shape[-1]
    n_experts, d_expert, _ = w_gate.shape
    n_tokens = batch * seq_len
    n_assign = n_tokens * top_k

    blk_m = 1024
    blk_n = 1024
    n_rows = n_assign + n_experts * blk_m

    x_flat = x.reshape(n_tokens, d_model)
    e_flat = expert_indices.reshape(n_assign).astype(jnp.int32)
    w_flat = routing_weights.reshape(n_assign).astype(jnp.float32)

    order = jnp.argsort(e_flat)
    e_sorted = jnp.take(e_flat, order)
    counts = jnp.bincount(e_flat, length=n_experts)
    starts = jnp.cumsum(counts) - counts
    pc = ((counts + blk_m - 1) // blk_m) * blk_m
    padded_starts = jnp.cumsum(pc) - pc
    p = jnp.arange(n_assign, dtype=jnp.int32)
    row_sorted = (padded_starts[e_sorted] + (p - starts[e_sorted])).astype(jnp.int32)
    w_row = jnp.zeros((n_rows,), jnp.float32).at[row_sorted].set(jnp.take(w_flat, order))
    nt = n_rows // blk_m
    tile_start = jnp.arange(nt, dtype=jnp.int32) * blk_m
    pcum = jnp.cumsum(pc)
    total_rows = pcum[-1]
    tile_valid = (tile_start < total_rows).astype(jnp.int32)
    n_valid = total_rows // blk_m
    te_raw = jnp.minimum(
        jnp.searchsorted(pcum, tile_start, side="right"), n_experts - 1
    ).astype(jnp.int32)
    last_te = jnp.take(te_raw, n_valid - 1)
    tile_expert = jnp.where(tile_valid == 1, te_raw, last_te)
    tile_mi = jnp.minimum(
        jnp.arange(nt, dtype=jnp.int32), (n_valid - 1).astype(jnp.int32)
    )
    row_by_a = jnp.zeros((n_assign,), jnp.int32).at[order].set(row_sorted)
    r0 = row_by_a[0::top_k]
    r1 = row_by_a[1::top_k]

    d_chunk = 32
    d_nc = n_tokens // NW // d_chunk
    r0_3 = r0.reshape(NW, d_nc, d_chunk)
    r1_3 = r1.reshape(NW, d_nc, d_chunk)
    xg = _make_dispatch(n_tokens, n_rows, d_model, d_chunk)(x_flat, r0_3, r1_3)

    w_row3 = w_row.reshape(nt, 1, blk_m)
    y = _make_gmm(n_rows, d_model, d_expert, n_experts, blk_m, blk_n)(
        tile_expert, tile_mi, tile_valid, xg, w_gate, w_up, w_down, w_row3
    )

    out = _make_combine(n_tokens, d_model, chunk=16)(y, r0, r1)
    return out.reshape(batch, seq_len, d_model)


# speedup vs baseline: 2.2814x; 1.0000x over previous
import functools

import jax
import jax.numpy as jnp
from jax import lax
from jax.experimental import pallas as pl
from jax.experimental.pallas import tpu as pltpu
from jax.experimental.pallas import tpu_sc as plsc

NW = 32


def _make_dispatch(n_tokens, n_rows, d_model, chunk):
    mesh = plsc.VectorSubcoreMesh(core_axis_name="c", subcore_axis_name="s")
    tok_per_w = n_tokens // NW
    nc = tok_per_w // chunk

    @functools.partial(
        pl.kernel,
        out_type=jax.ShapeDtypeStruct((n_rows, d_model), jnp.float32),
        mesh=mesh,
        scratch_types=[
            pltpu.VMEM((nc, chunk), jnp.int32),
            pltpu.VMEM((nc, chunk), jnp.int32),
            pltpu.VMEM((2, chunk, d_model), jnp.float32),
            pltpu.SemaphoreType.DMA,
            pltpu.SemaphoreType.DMA,
            pltpu.SemaphoreType.DMA,
            pltpu.SemaphoreType.DMA,
            pltpu.SemaphoreType.DMA,
            pltpu.SemaphoreType.DMA,
        ],
    )
    def dispatch_k(x_hbm, r0_hbm, r1_hbm, out_hbm, i0_v, i1_v, buf_v,
                   l0, l1, sa0, sa1, sb0, sb1):
        wid = lax.axis_index("s") * 2 + lax.axis_index("c")
        base = wid * tok_per_w
        lin = (l0, l1)
        sa = (sa0, sa1)
        sb = (sb0, sb1)
        pltpu.sync_copy(r0_hbm.at[wid], i0_v)
        pltpu.sync_copy(r1_hbm.at[wid], i1_v)

        lh = [None] * nc
        sha = [None] * nc
        shb = [None] * nc
        lh[0] = pltpu.async_copy(
            x_hbm.at[pl.ds(base, chunk)], buf_v.at[0], lin[0]
        )
        for c in range(nc):
            b = c % 2
            if c + 1 < nc:
                b2 = (c + 1) % 2
                if c - 1 >= 0:
                    sha[c - 1].wait()
                    shb[c - 1].wait()
                lh[c + 1] = pltpu.async_copy(
                    x_hbm.at[pl.ds(base + (c + 1) * chunk, chunk)],
                    buf_v.at[b2],
                    lin[b2],
                )
            lh[c].wait()
            sha[c] = pltpu.async_copy(buf_v.at[b], out_hbm.at[i0_v.at[c]], sa[b])
            shb[c] = pltpu.async_copy(buf_v.at[b], out_hbm.at[i1_v.at[c]], sb[b])
        if nc >= 2:
            sha[nc - 2].wait()
            shb[nc - 2].wait()
        sha[nc - 1].wait()
        shb[nc - 1].wait()

    return dispatch_k


def _make_combine(n_tokens, d_model, chunk):
    mesh = plsc.VectorSubcoreMesh(core_axis_name="c", subcore_axis_name="s")
    tok_per_w = n_tokens // NW
    nc = tok_per_w // chunk
    lanes_per_row = d_model // 16

    nbuf = 3

    @functools.partial(
        pl.kernel,
        out_type=jax.ShapeDtypeStruct((n_tokens, d_model), jnp.float32),
        mesh=mesh,
        scratch_types=[
            pltpu.VMEM((tok_per_w,), jnp.int32),
            pltpu.VMEM((tok_per_w,), jnp.int32),
            pltpu.VMEM((nbuf, chunk, d_model), jnp.float32),
            pltpu.VMEM((nbuf, chunk, d_model), jnp.float32),
        ]
        + [pltpu.SemaphoreType.DMA] * (3 * nbuf),
    )
    def combine_k(y_hbm, r0_hbm, r1_hbm, out_hbm, i0_v, i1_v, a_v, b_v, *sems):
        wid = lax.axis_index("s") * 2 + lax.axis_index("c")
        base = wid * tok_per_w
        ga = sems[0:nbuf]
        gb = sems[nbuf : 2 * nbuf]
        ss = sems[2 * nbuf : 3 * nbuf]
        pltpu.sync_copy(r0_hbm.at[pl.ds(base, tok_per_w)], i0_v)
        pltpu.sync_copy(r1_hbm.at[pl.ds(base, tok_per_w)], i1_v)

        def start(c):
            b = c % nbuf
            ha = pltpu.async_copy(
                y_hbm.at[i0_v.at[pl.ds(c * chunk, chunk)]], a_v.at[b], ga[b]
            )
            hb = pltpu.async_copy(
                y_hbm.at[i1_v.at[pl.ds(c * chunk, chunk)]], b_v.at[b], gb[b]
            )
            return ha, hb

        hs = [None] * nc
        st = [None] * nc
        for c in range(min(nbuf - 1, nc)):
            hs[c] = start(c)
        for c in range(nc):
            b = c % nbuf
            if c + nbuf - 1 < nc:
                if c - 1 >= 0:
                    st[c - 1].wait()
                hs[c + nbuf - 1] = start(c + nbuf - 1)
            hs[c][0].wait()
            hs[c][1].wait()

            def row_add(r, carry):
                def col_add(k, c3):
                    for u in range(8):
                        sl = pl.ds((k * 8 + u) * 16, 16)
                        a_v[b, r, sl] = a_v[b, r, sl] + b_v[b, r, sl]
                    return c3

                lax.fori_loop(0, lanes_per_row // 8, col_add, 0)
                return carry

            lax.fori_loop(0, chunk, row_add, 0)
            st[c] = pltpu.async_copy(
                a_v.at[b], out_hbm.at[pl.ds(base + c * chunk, chunk)], ss[b]
            )
        for c in range(max(0, nc - nbuf), nc):
            st[c].wait()

    return combine_k


def _gmm_body(te_ref, mi_ref, valid_ref, xg_ref, wg_ref, wu_ref, wd_ref,
              wrow_ref, y_ref):
    i = pl.program_id(0)
    j = pl.program_id(1)

    @pl.when(valid_ref[i] != 0)
    def _():
        x = xg_ref[...]
        g = jnp.dot(x, wg_ref[0].T, preferred_element_type=jnp.float32)
        u = jnp.dot(x, wu_ref[0].T, preferred_element_type=jnp.float32)
        h = (g * jax.nn.sigmoid(g)) * u
        yj = jnp.dot(h, wd_ref[0].T, preferred_element_type=jnp.float32)
        yj = yj * wrow_ref[0, 0, :][:, None]

        @pl.when(j == 0)
        def _():
            y_ref[...] = jnp.zeros_like(y_ref)

        y_ref[...] += yj


def _make_gmm(n_rows, d_model, d_expert, n_experts, blk_m, blk_n):
    nt = n_rows // blk_m
    nb = d_expert // blk_n
    grid_spec = pltpu.PrefetchScalarGridSpec(
        num_scalar_prefetch=3,
        grid=(nt, nb),
        in_specs=[
            pl.BlockSpec((blk_m, d_model), lambda i, j, te, mi, v: (mi[i], 0)),
            pl.BlockSpec((1, blk_n, d_model), lambda i, j, te, mi, v: (te[i], j, 0)),
            pl.BlockSpec((1, blk_n, d_model), lambda i, j, te, mi, v: (te[i], j, 0)),
            pl.BlockSpec((1, d_model, blk_n), lambda i, j, te, mi, v: (te[i], 0, j)),
            pl.BlockSpec((1, 1, blk_m), lambda i, j, te, mi, v: (mi[i], 0, 0)),
        ],
        out_specs=pl.BlockSpec((blk_m, d_model), lambda i, j, te, mi, v: (mi[i], 0)),
    )
    return pl.pallas_call(
        _gmm_body,
        grid_spec=grid_spec,
        out_shape=jax.ShapeDtypeStruct((n_rows, d_model), jnp.float32),
        compiler_params=pltpu.CompilerParams(
            dimension_semantics=("arbitrary", "arbitrary"),
        ),
    )


def kernel(x, routing_weights, expert_indices, w_gate, w_up, w_down):
    batch, seq_len, d_model = x.shape
    top_k = expert_indices.shape[-1]
    n_experts, d_expert, _ = w_gate.shape
    n_tokens = batch * seq_len
    n_assign = n_tokens * top_k

    blk_m = 1024
    blk_n = 1024
    n_rows = n_assign + n_experts * blk_m

    x_flat = x.reshape(n_tokens, d_model)
    e_flat = expert_indices.reshape(n_assign).astype(jnp.int32)
    w_flat = routing_weights.reshape(n_assign).astype(jnp.float32)

    order = jnp.argsort(e_flat)
    e_sorted = jnp.take(e_flat, order)
    counts = jnp.bincount(e_flat, length=n_experts)
    starts = jnp.cumsum(counts) - counts
    pc = ((counts + blk_m - 1) // blk_m) * blk_m
    padded_starts = jnp.cumsum(pc) - pc
    p = jnp.arange(n_assign, dtype=jnp.int32)
    row_sorted = (padded_starts[e_sorted] + (p - starts[e_sorted])).astype(jnp.int32)
    w_row = jnp.zeros((n_rows,), jnp.float32).at[row_sorted].set(jnp.take(w_flat, order))
    nt = n_rows // blk_m
    tile_start = jnp.arange(nt, dtype=jnp.int32) * blk_m
    pcum = jnp.cumsum(pc)
    total_rows = pcum[-1]
    tile_valid = (tile_start < total_rows).astype(jnp.int32)
    n_valid = total_rows // blk_m
    te_raw = jnp.minimum(
        jnp.searchsorted(pcum, tile_start, side="right"), n_experts - 1
    ).astype(jnp.int32)
    last_te = jnp.take(te_raw, n_valid - 1)
    tile_expert = jnp.where(tile_valid == 1, te_raw, last_te)
    tile_mi = jnp.minimum(
        jnp.arange(nt, dtype=jnp.int32), (n_valid - 1).astype(jnp.int32)
    )
    row_by_a = jnp.zeros((n_assign,), jnp.int32).at[order].set(row_sorted)
    r0 = row_by_a[0::top_k]
    r1 = row_by_a[1::top_k]

    d_chunk = 32
    d_nc = n_tokens // NW // d_chunk
    r0_3 = r0.reshape(NW, d_nc, d_chunk)
    r1_3 = r1.reshape(NW, d_nc, d_chunk)
    xg = _make_dispatch(n_tokens, n_rows, d_model, d_chunk)(x_flat, r0_3, r1_3)

    w_row3 = w_row.reshape(nt, 1, blk_m)
    y = _make_gmm(n_rows, d_model, d_expert, n_experts, blk_m, blk_n)(
        tile_expert, tile_mi, tile_valid, xg, w_gate, w_up, w_down, w_row3
    )

    out = _make_combine(n_tokens, d_model, chunk=16)(y, r0, r1)
    return out.reshape(batch, seq_len, d_model)
